# baseline (speedup 1.0000x reference)
import functools

import jax
import jax.numpy as jnp
from jax import lax
from jax.experimental import pallas as pl
from jax.experimental.pallas import tpu as pltpu
from jax.experimental.pallas import tpu_sc as plsc


def _make_sc_gather(N, K, NC, NS, L):
    NW = NC * NS
    CHN = 256
    NPAD = (N + CHN - 1) // CHN * CHN
    NCHT = NPAD // CHN
    NCHF = N // CHN
    assert K == L and N % 8 == 0
    TRIPS = (NCHT + NW - 1) // NW

    mesh = plsc.VectorSubcoreMesh(core_axis_name="c", subcore_axis_name="s")

    @functools.partial(
        pl.kernel,
        out_type=(jax.ShapeDtypeStruct((K, NPAD), jnp.int32),
                  jax.ShapeDtypeStruct((K, NPAD), jnp.float32)),
        mesh=mesh,
        compiler_params=pltpu.CompilerParams(needs_layout_passes=False),
        scratch_types=[
            pltpu.VMEM((NPAD,), jnp.int32),
            pltpu.VMEM((NPAD,), jnp.float32),
            pltpu.VMEM((2, K, CHN), jnp.int32),
            pltpu.VMEM((2, K, CHN), jnp.int32),
            pltpu.VMEM((2, K, CHN), jnp.float32),
            pltpu.SemaphoreType.DMA,
            pltpu.SemaphoreType.DMA,
        ],
    )
    def k(xyp_hbm, z_hbm, nbr_hbm, nbrtail_hbm, gxy_hbm, gz_hbm,
          tblxy, tblz, idx2, gxy2, gz2, sin, sout):
        wid = lax.axis_index("s") * NC + lax.axis_index("c")

        def issue_idx(ch, par):
            @pl.when(ch < NCHF)
            def _():
                pltpu.async_copy(nbr_hbm.at[:, pl.ds(ch * CHN, CHN)],
                                 idx2.at[par], sin)

            @pl.when(ch == NCHF)
            def _():
                pltpu.async_copy(nbrtail_hbm, idx2.at[par], sin)

        pltpu.sync_copy(xyp_hbm.at[pl.ds(0, N)], tblxy.at[pl.ds(0, N)])
        pltpu.sync_copy(z_hbm.at[pl.ds(0, N)], tblz.at[pl.ds(0, N)])
        issue_idx(wid, 0)

        def pair_body(i2, _):
            for par in (0, 1):
                r = i2 * 2 + par
                ch = wid + r * NW

                @pl.when(ch < NCHT)
                def _(r=r, ch=ch, par=par):
                    pltpu.make_async_copy(
                        nbr_hbm.at[:, pl.ds(0, CHN)], idx2.at[par], sin
                    ).wait()
                    issue_idx(ch + NW, par ^ 1)

                    @pl.when(r >= 2)
                    def _():
                        pltpu.make_async_copy(
                            gxy_hbm.at[:, pl.ds(0, CHN)],
                            gxy2.at[par], sout).wait()
                        pltpu.make_async_copy(
                            gz_hbm.at[:, pl.ds(0, CHN)],
                            gz2.at[par], sout).wait()

                    @plsc.parallel_loop(0, CHN // L, unroll=4)
                    def jj_body(jj):
                        for kk in range(K):
                            idxv = idx2[par, kk, pl.ds(jj * L, L)]
                            gxy2[par, kk, pl.ds(jj * L, L)] = (
                                plsc.load_gather(tblxy, [idxv]))
                            gz2[par, kk, pl.ds(jj * L, L)] = (
                                plsc.load_gather(tblz, [idxv]))

                    n0 = ch * CHN
                    pltpu.async_copy(gxy2.at[par],
                                     gxy_hbm.at[:, pl.ds(n0, CHN)], sout)
                    pltpu.async_copy(gz2.at[par],
                                     gz_hbm.at[:, pl.ds(n0, CHN)], sout)

            return 0

        lax.fori_loop(0, (TRIPS + 1) // 2, pair_body, 0)

        tw = (NCHT - wid + NW - 1) // NW
        for thresh in (1, 2):
            @pl.when(tw >= thresh)
            def _():
                pltpu.make_async_copy(
                    gxy_hbm.at[:, pl.ds(0, CHN)], gxy2.at[0], sout).wait()
                pltpu.make_async_copy(
                    gz_hbm.at[:, pl.ds(0, CHN)], gz2.at[0], sout).wait()

    return k, NPAD, NCHF * CHN


def _make_tc_assemble(B, N, K, NPAD, b, first):
    BN = 8192
    NB = (N + BN - 1) // BN

    def body(*refs):
        if first:
            xyz3_ref, gxy_ref, gz_ref, dist_ref, out_ref = refs
        else:
            _, xyz3_ref, gxy_ref, gz_ref, dist_ref, out_ref = refs
        own = xyz3_ref[...]
        pxy = gxy_ref[...]
        nbs = (
            lax.bitcast_convert_type(pxy & jnp.int32(-65536), jnp.float32),
            lax.bitcast_convert_type(pxy << 16, jnp.float32),
            gz_ref[...],
        )
        for c in range(3):
            bc = jnp.broadcast_to(own[c][None, :], (K, BN))
            out_ref[0, c] = bc
            out_ref[0, 3 + c] = nbs[c]
            out_ref[0, 6 + c] = bc - nbs[c]
        out_ref[0, 9] = dist_ref[...]

    specs = [
        pl.BlockSpec((3, BN), lambda i: (0, i)),
        pl.BlockSpec((K, BN), lambda i: (0, i)),
        pl.BlockSpec((K, BN), lambda i: (0, i)),
        pl.BlockSpec((K, BN), lambda i: (0, i)),
    ]
    if not first:
        specs.insert(0, pl.BlockSpec(memory_space=pl.ANY))
    return pl.pallas_call(
        body,
        grid=(NB,),
        in_specs=specs,
        out_specs=pl.BlockSpec((1, 10, K, BN), lambda i: (b, 0, 0, i)),
        out_shape=jax.ShapeDtypeStruct((B, 10, K, N), jnp.float32),
        input_output_aliases={} if first else {0: 0},
    )


def kernel(xyz, neighbors, distances):
    B, N, K = neighbors.shape
    info = plsc.get_sparse_core_info()
    sc_gather, NPAD, T0 = _make_sc_gather(
        N, K, info.num_cores, info.num_subcores, info.num_lanes)

    xi = lax.bitcast_convert_type(xyz[:, :, 0], jnp.uint32)
    yi = lax.bitcast_convert_type(xyz[:, :, 1], jnp.uint32)
    xyp = lax.bitcast_convert_type(
        ((xi + 0x8000) & jnp.uint32(0xFFFF0000)) | ((yi + 0x8000) >> 16),
        jnp.int32)
    zmat = xyz[:, :, 2]

    xyz3 = jnp.transpose(xyz, (0, 2, 1))
    nbr_t = jnp.transpose(neighbors.astype(jnp.int32), (0, 2, 1))
    nbr_tail = jnp.pad(nbr_t[:, :, T0:], ((0, 0), (0, 0), (0, NPAD - N)))
    dist_t = jnp.transpose(distances, (0, 2, 1))

    gs = [sc_gather(xyp[b], zmat[b], nbr_t[b], nbr_tail[b]) for b in range(B)]
    acc = None
    for b in range(B):
        gxy, gz = gs[b]
        tc = _make_tc_assemble(B, N, K, NPAD, b, first=(b == 0))
        args = (xyz3[b], gxy, gz, dist_t[b])
        acc = tc(*args) if acc is None else tc(acc, *args)
    return jnp.transpose(acc, (0, 1, 3, 2))


# speedup vs baseline: 1.1648x; 1.1648x over previous
import functools

import jax
import jax.numpy as jnp
from jax import lax
from jax.experimental import pallas as pl
from jax.experimental.pallas import tpu as pltpu
from jax.experimental.pallas import tpu_sc as plsc


def _make_sc_gather(B, N, K, NC, NS, L):
    NW = NC * NS
    CHN = 256
    NPAD = (N + CHN - 1) // CHN * CHN
    NCHT = NPAD // CHN
    NCHF = N // CHN
    assert K == L and N % 8 == 0
    TRIPS = (NCHT + NW - 1) // NW

    mesh = plsc.VectorSubcoreMesh(core_axis_name="c", subcore_axis_name="s")

    @functools.partial(
        pl.kernel,
        out_type=(jax.ShapeDtypeStruct((B, K, NPAD), jnp.int32),
                  jax.ShapeDtypeStruct((B, K, NPAD), jnp.float32)),
        mesh=mesh,
        compiler_params=pltpu.CompilerParams(needs_layout_passes=False),
        scratch_types=[
            pltpu.VMEM((NPAD,), jnp.int32),
            pltpu.VMEM((NPAD,), jnp.float32),
            pltpu.VMEM((2, K, CHN), jnp.int32),
            pltpu.VMEM((2, K, CHN), jnp.int32),
            pltpu.VMEM((2, K, CHN), jnp.float32),
            pltpu.SemaphoreType.DMA,
            pltpu.SemaphoreType.DMA,
        ],
    )
    def k(xyp_hbm, z_hbm, nbr_hbm, nbrtail_hbm, gxy_hbm, gz_hbm,
          tblxy, tblz, idx2, gxy2, gz2, sin, sout):
        wid = lax.axis_index("s") * NC + lax.axis_index("c")

        def issue_idx(b, ch, par):
            @pl.when(ch < NCHF)
            def _():
                pltpu.async_copy(nbr_hbm.at[b, :, pl.ds(ch * CHN, CHN)],
                                 idx2.at[par], sin)

            @pl.when(ch == NCHF)
            def _():
                pltpu.async_copy(nbrtail_hbm.at[b], idx2.at[par], sin)

        def batch_body(b, _):
            pltpu.sync_copy(xyp_hbm.at[pl.ds(b * N, N)], tblxy.at[pl.ds(0, N)])
            pltpu.sync_copy(z_hbm.at[pl.ds(b * N, N)], tblz.at[pl.ds(0, N)])
            issue_idx(b, wid, 0)

            def pair_body(i2, _):
                for par in (0, 1):
                    r = i2 * 2 + par
                    ch = wid + r * NW

                    @pl.when(ch < NCHT)
                    def _(r=r, ch=ch, par=par):
                        pltpu.make_async_copy(
                            nbr_hbm.at[b, :, pl.ds(0, CHN)], idx2.at[par], sin
                        ).wait()
                        issue_idx(b, ch + NW, par ^ 1)

                        @pl.when(r >= 2)
                        def _():
                            pltpu.make_async_copy(
                                gxy_hbm.at[b, :, pl.ds(0, CHN)],
                                gxy2.at[par], sout).wait()
                            pltpu.make_async_copy(
                                gz_hbm.at[b, :, pl.ds(0, CHN)],
                                gz2.at[par], sout).wait()

                        @plsc.parallel_loop(0, CHN // L, unroll=8)
                        def jj_body(jj):
                            for kk in range(K):
                                idxv = idx2[par, kk, pl.ds(jj * L, L)]
                                gxy2[par, kk, pl.ds(jj * L, L)] = (
                                    plsc.load_gather(tblxy, [idxv]))
                                gz2[par, kk, pl.ds(jj * L, L)] = (
                                    plsc.load_gather(tblz, [idxv]))

                        n0 = ch * CHN
                        pltpu.async_copy(gxy2.at[par],
                                         gxy_hbm.at[b, :, pl.ds(n0, CHN)], sout)
                        pltpu.async_copy(gz2.at[par],
                                         gz_hbm.at[b, :, pl.ds(n0, CHN)], sout)

                return 0

            lax.fori_loop(0, (TRIPS + 1) // 2, pair_body, 0)

            tw = (NCHT - wid + NW - 1) // NW
            for thresh in (1, 2):
                @pl.when(tw >= thresh)
                def _():
                    pltpu.make_async_copy(
                        gxy_hbm.at[b, :, pl.ds(0, CHN)], gxy2.at[0], sout).wait()
                    pltpu.make_async_copy(
                        gz_hbm.at[b, :, pl.ds(0, CHN)], gz2.at[0], sout).wait()

            return 0

        lax.fori_loop(0, B, batch_body, 0)

    return k, NPAD, NCHF * CHN


def _make_tc_assemble(B, N, K, NPAD):
    BN = 8192
    NB = (N + BN - 1) // BN

    def body(xyz3_ref, gxy_ref, gz_ref, dist_ref, out_ref):
        own = xyz3_ref[0]
        pxy = gxy_ref[0]
        nbs = (
            lax.bitcast_convert_type(pxy & jnp.int32(-65536), jnp.float32),
            lax.bitcast_convert_type(pxy << 16, jnp.float32),
            gz_ref[0],
        )
        for c in range(3):
            bc = jnp.broadcast_to(own[c][None, :], (K, BN))
            out_ref[0, c] = bc
            out_ref[0, 3 + c] = nbs[c]
            out_ref[0, 6 + c] = bc - nbs[c]
        out_ref[0, 9] = dist_ref[0]

    return pl.pallas_call(
        body,
        grid=(B, NB),
        in_specs=[
            pl.BlockSpec((1, 3, BN), lambda b, i: (b, 0, i)),
            pl.BlockSpec((1, K, BN), lambda b, i: (b, 0, i)),
            pl.BlockSpec((1, K, BN), lambda b, i: (b, 0, i)),
            pl.BlockSpec((1, K, BN), lambda b, i: (b, 0, i)),
        ],
        out_specs=pl.BlockSpec((1, 10, K, BN), lambda b, i: (b, 0, 0, i)),
        out_shape=jax.ShapeDtypeStruct((B, 10, K, N), jnp.float32),
    )


def kernel(xyz, neighbors, distances):
    B, N, K = neighbors.shape
    info = plsc.get_sparse_core_info()
    sc_gather, NPAD, T0 = _make_sc_gather(
        B, N, K, info.num_cores, info.num_subcores, info.num_lanes)

    xi = lax.bitcast_convert_type(xyz[:, :, 0], jnp.uint32)
    yi = lax.bitcast_convert_type(xyz[:, :, 1], jnp.uint32)
    xyp = lax.bitcast_convert_type(
        ((xi + 0x8000) & jnp.uint32(0xFFFF0000)) | ((yi + 0x8000) >> 16),
        jnp.int32).reshape(B * N)
    zflat = xyz[:, :, 2].reshape(B * N)

    xyz3 = jnp.transpose(xyz, (0, 2, 1))
    nbr_t = jnp.transpose(neighbors.astype(jnp.int32), (0, 2, 1))
    nbr_tail = jnp.pad(nbr_t[:, :, T0:], ((0, 0), (0, 0), (0, NPAD - N)))
    dist_t = jnp.transpose(distances, (0, 2, 1))

    gxy, gz = sc_gather(xyp, zflat, nbr_t, nbr_tail)
    out_t = _make_tc_assemble(B, N, K, NPAD)(xyz3, gxy, gz, dist_t)
    return jnp.transpose(out_t, (0, 1, 3, 2))


# speedup vs baseline: 1.1941x; 1.0252x over previous
import functools

import jax
import jax.numpy as jnp
from jax import lax
from jax.experimental import pallas as pl
from jax.experimental.pallas import tpu as pltpu
from jax.experimental.pallas import tpu_sc as plsc


def _make_sc_gather(B, N, K, NC, NS, L):
    NW = NC * NS
    CHN = 256
    NPAD = (N + CHN - 1) // CHN * CHN
    NCHT = NPAD // CHN
    NCHF = N // CHN
    assert K == L and N % 8 == 0
    TRIPS = (NCHT + NW - 1) // NW

    mesh = plsc.VectorSubcoreMesh(core_axis_name="c", subcore_axis_name="s")

    @functools.partial(
        pl.kernel,
        out_type=(jax.ShapeDtypeStruct((B, K, NPAD), jnp.int32),
                  jax.ShapeDtypeStruct((B, K, NPAD), jnp.float32)),
        mesh=mesh,
        compiler_params=pltpu.CompilerParams(needs_layout_passes=False),
        scratch_types=[
            pltpu.VMEM((NPAD,), jnp.int32),
            pltpu.VMEM((NPAD,), jnp.float32),
            pltpu.VMEM((2, K, CHN), jnp.int32),
            pltpu.VMEM((2, K, CHN), jnp.int32),
            pltpu.VMEM((2, K, CHN), jnp.float32),
            pltpu.SemaphoreType.DMA,
            pltpu.SemaphoreType.DMA,
        ],
    )
    def k(xyp_hbm, z_hbm, nbr_hbm, nbrtail_hbm, gxy_hbm, gz_hbm,
          tblxy, tblz, idx2, gxy2, gz2, sin, sout):
        wid = lax.axis_index("s") * NC + lax.axis_index("c")

        def issue_idx(b, ch, par):
            @pl.when(ch < NCHF)
            def _():
                pltpu.async_copy(nbr_hbm.at[b, :, pl.ds(ch * CHN, CHN)],
                                 idx2.at[par], sin)

            @pl.when(ch == NCHF)
            def _():
                pltpu.async_copy(nbrtail_hbm.at[b], idx2.at[par], sin)

        def batch_body(b, _):
            pltpu.sync_copy(xyp_hbm.at[pl.ds(b * N, N)], tblxy.at[pl.ds(0, N)])
            pltpu.sync_copy(z_hbm.at[pl.ds(b * N, N)], tblz.at[pl.ds(0, N)])
            issue_idx(b, wid, 0)

            def pair_body(i2, _):
                for par in (0, 1):
                    r = i2 * 2 + par
                    ch = wid + r * NW

                    @pl.when(ch < NCHT)
                    def _(r=r, ch=ch, par=par):
                        pltpu.make_async_copy(
                            nbr_hbm.at[b, :, pl.ds(0, CHN)], idx2.at[par], sin
                        ).wait()
                        issue_idx(b, ch + NW, par ^ 1)

                        @pl.when(r >= 2)
                        def _():
                            pltpu.make_async_copy(
                                gxy_hbm.at[b, :, pl.ds(0, CHN)],
                                gxy2.at[par], sout).wait()
                            pltpu.make_async_copy(
                                gz_hbm.at[b, :, pl.ds(0, CHN)],
                                gz2.at[par], sout).wait()

                        @plsc.parallel_loop(0, CHN // L, unroll=8)
                        def jj_body(jj):
                            for kk in range(K):
                                idxv = idx2[par, kk, pl.ds(jj * L, L)]
                                gxy2[par, kk, pl.ds(jj * L, L)] = (
                                    plsc.load_gather(tblxy, [idxv]))
                                gz2[par, kk, pl.ds(jj * L, L)] = (
                                    plsc.load_gather(tblz, [idxv]))

                        n0 = ch * CHN
                        pltpu.async_copy(gxy2.at[par],
                                         gxy_hbm.at[b, :, pl.ds(n0, CHN)], sout)
                        pltpu.async_copy(gz2.at[par],
                                         gz_hbm.at[b, :, pl.ds(n0, CHN)], sout)

                return 0

            lax.fori_loop(0, (TRIPS + 1) // 2, pair_body, 0)

            tw = (NCHT - wid + NW - 1) // NW
            for thresh in (1, 2):
                @pl.when(tw >= thresh)
                def _():
                    pltpu.make_async_copy(
                        gxy_hbm.at[b, :, pl.ds(0, CHN)], gxy2.at[0], sout).wait()
                    pltpu.make_async_copy(
                        gz_hbm.at[b, :, pl.ds(0, CHN)], gz2.at[0], sout).wait()

            return 0

        lax.fori_loop(0, B, batch_body, 0)

    return k, NPAD, NCHF * CHN


def _make_tc_assemble(B, N, K, NPAD):
    BN = 16384
    NB = (N + BN - 1) // BN

    def body(xyz3_ref, gxy_ref, gz_ref, dist_ref, out_ref):
        own = xyz3_ref[0]
        pxy = gxy_ref[0]
        nbs = (
            lax.bitcast_convert_type(pxy & jnp.int32(-65536), jnp.float32),
            lax.bitcast_convert_type(pxy << 16, jnp.float32),
            gz_ref[0],
        )
        for c in range(3):
            bc = jnp.broadcast_to(own[c][None, :], (K, BN))
            out_ref[0, c] = bc
            out_ref[0, 3 + c] = nbs[c]
            out_ref[0, 6 + c] = bc - nbs[c]
        out_ref[0, 9] = dist_ref[0]

    return pl.pallas_call(
        body,
        grid=(B, NB),
        in_specs=[
            pl.BlockSpec((1, 3, BN), lambda b, i: (b, 0, i)),
            pl.BlockSpec((1, K, BN), lambda b, i: (b, 0, i)),
            pl.BlockSpec((1, K, BN), lambda b, i: (b, 0, i)),
            pl.BlockSpec((1, K, BN), lambda b, i: (b, 0, i)),
        ],
        out_specs=pl.BlockSpec((1, 10, K, BN), lambda b, i: (b, 0, 0, i)),
        out_shape=jax.ShapeDtypeStruct((B, 10, K, N), jnp.float32),
    )


def kernel(xyz, neighbors, distances):
    B, N, K = neighbors.shape
    info = plsc.get_sparse_core_info()
    sc_gather, NPAD, T0 = _make_sc_gather(
        B, N, K, info.num_cores, info.num_subcores, info.num_lanes)

    xi = lax.bitcast_convert_type(xyz[:, :, 0], jnp.uint32)
    yi = lax.bitcast_convert_type(xyz[:, :, 1], jnp.uint32)
    xyp = lax.bitcast_convert_type(
        ((xi + 0x8000) & jnp.uint32(0xFFFF0000)) | ((yi + 0x8000) >> 16),
        jnp.int32).reshape(B * N)
    zflat = xyz[:, :, 2].reshape(B * N)

    xyz3 = jnp.transpose(xyz, (0, 2, 1))
    nbr_t = jnp.transpose(neighbors.astype(jnp.int32), (0, 2, 1))
    nbr_tail = jnp.pad(nbr_t[:, :, T0:], ((0, 0), (0, 0), (0, NPAD - N)))
    dist_t = jnp.transpose(distances, (0, 2, 1))

    gxy, gz = sc_gather(xyp, zflat, nbr_t, nbr_tail)
    out_t = _make_tc_assemble(B, N, K, NPAD)(xyz3, gxy, gz, dist_t)
    return jnp.transpose(out_t, (0, 1, 3, 2))


# speedup vs baseline: 1.2063x; 1.0102x over previous
import functools

import jax
import jax.numpy as jnp
from jax import lax
from jax.experimental import pallas as pl
from jax.experimental.pallas import tpu as pltpu
from jax.experimental.pallas import tpu_sc as plsc


def _make_sc_gather(B, N, K, NC, NS, L):
    NW = NC * NS
    CHN = 256
    NPAD = (N + CHN - 1) // CHN * CHN
    NCHT = NPAD // CHN
    NCHF = N // CHN
    assert K == L and N % 8 == 0
    TRIPS = (NCHT + NW - 1) // NW

    mesh = plsc.VectorSubcoreMesh(core_axis_name="c", subcore_axis_name="s")

    @functools.partial(
        pl.kernel,
        out_type=(jax.ShapeDtypeStruct((B, K, NPAD), jnp.int32),
                  jax.ShapeDtypeStruct((B, K, NPAD), jnp.float32)),
        mesh=mesh,
        compiler_params=pltpu.CompilerParams(needs_layout_passes=False),
        scratch_types=[
            pltpu.VMEM((NPAD,), jnp.int32),
            pltpu.VMEM((NPAD,), jnp.float32),
            pltpu.VMEM((2, K, CHN), jnp.int32),
            pltpu.VMEM((2, K, CHN), jnp.int32),
            pltpu.VMEM((2, K, CHN), jnp.float32),
            pltpu.SemaphoreType.DMA,
            pltpu.SemaphoreType.DMA,
        ],
    )
    def k(xyp_hbm, z_hbm, nbr_hbm, nbrtail_hbm, gxy_hbm, gz_hbm,
          tblxy, tblz, idx2, gxy2, gz2, sin, sout):
        wid = lax.axis_index("s") * NC + lax.axis_index("c")

        def issue_idx(b, ch, par):
            @pl.when(ch < NCHF)
            def _():
                pltpu.async_copy(nbr_hbm.at[b, :, pl.ds(ch * CHN, CHN)],
                                 idx2.at[par], sin)

            @pl.when(ch == NCHF)
            def _():
                pltpu.async_copy(nbrtail_hbm.at[b], idx2.at[par], sin)

        def batch_body(b, _):
            pltpu.sync_copy(xyp_hbm.at[pl.ds(b * N, N)], tblxy.at[pl.ds(0, N)])
            pltpu.sync_copy(z_hbm.at[pl.ds(b * N, N)], tblz.at[pl.ds(0, N)])
            issue_idx(b, wid, 0)

            def pair_body(i2, _):
                for par in (0, 1):
                    r = i2 * 2 + par
                    ch = wid + r * NW

                    @pl.when(ch < NCHT)
                    def _(r=r, ch=ch, par=par):
                        pltpu.make_async_copy(
                            nbr_hbm.at[b, :, pl.ds(0, CHN)], idx2.at[par], sin
                        ).wait()
                        issue_idx(b, ch + NW, par ^ 1)

                        @pl.when(r >= 2)
                        def _():
                            pltpu.make_async_copy(
                                gxy_hbm.at[b, :, pl.ds(0, CHN)],
                                gxy2.at[par], sout).wait()
                            pltpu.make_async_copy(
                                gz_hbm.at[b, :, pl.ds(0, CHN)],
                                gz2.at[par], sout).wait()

                        @plsc.parallel_loop(0, CHN // L, unroll=8)
                        def jj_body(jj):
                            for kk in range(K):
                                idxv = idx2[par, kk, pl.ds(jj * L, L)]
                                gxy2[par, kk, pl.ds(jj * L, L)] = (
                                    plsc.load_gather(tblxy, [idxv]))
                                gz2[par, kk, pl.ds(jj * L, L)] = (
                                    plsc.load_gather(tblz, [idxv]))

                        n0 = ch * CHN
                        pltpu.async_copy(gxy2.at[par],
                                         gxy_hbm.at[b, :, pl.ds(n0, CHN)], sout)
                        pltpu.async_copy(gz2.at[par],
                                         gz_hbm.at[b, :, pl.ds(n0, CHN)], sout)

                return 0

            lax.fori_loop(0, (TRIPS + 1) // 2, pair_body, 0)

            tw = (NCHT - wid + NW - 1) // NW
            for thresh in (1, 2):
                @pl.when(tw >= thresh)
                def _():
                    pltpu.make_async_copy(
                        gxy_hbm.at[b, :, pl.ds(0, CHN)], gxy2.at[0], sout).wait()
                    pltpu.make_async_copy(
                        gz_hbm.at[b, :, pl.ds(0, CHN)], gz2.at[0], sout).wait()

            return 0

        lax.fori_loop(0, B, batch_body, 0)

    return k, NPAD, NCHF * CHN


def _make_tc_assemble(B, N, K, NPAD):
    BN = 25088
    NB = (N + BN - 1) // BN

    def body(xyz3_ref, gxy_ref, gz_ref, dist_ref, out_ref):
        own = xyz3_ref[0]
        pxy = gxy_ref[0]
        nbs = (
            lax.bitcast_convert_type(pxy & jnp.int32(-65536), jnp.float32),
            lax.bitcast_convert_type(pxy << 16, jnp.float32),
            gz_ref[0],
        )
        for c in range(3):
            bc = jnp.broadcast_to(own[c][None, :], (K, BN))
            out_ref[0, c] = bc
            out_ref[0, 3 + c] = nbs[c]
            out_ref[0, 6 + c] = bc - nbs[c]
        out_ref[0, 9] = dist_ref[0]

    return pl.pallas_call(
        body,
        grid=(B, NB),
        in_specs=[
            pl.BlockSpec((1, 3, BN), lambda b, i: (b, 0, i)),
            pl.BlockSpec((1, K, BN), lambda b, i: (b, 0, i)),
            pl.BlockSpec((1, K, BN), lambda b, i: (b, 0, i)),
            pl.BlockSpec((1, K, BN), lambda b, i: (b, 0, i)),
        ],
        out_specs=pl.BlockSpec((1, 10, K, BN), lambda b, i: (b, 0, 0, i)),
        out_shape=jax.ShapeDtypeStruct((B, 10, K, N), jnp.float32),
    )


def kernel(xyz, neighbors, distances):
    B, N, K = neighbors.shape
    info = plsc.get_sparse_core_info()
    sc_gather, NPAD, T0 = _make_sc_gather(
        B, N, K, info.num_cores, info.num_subcores, info.num_lanes)

    xi = lax.bitcast_convert_type(xyz[:, :, 0], jnp.uint32)
    yi = lax.bitcast_convert_type(xyz[:, :, 1], jnp.uint32)
    xyp = lax.bitcast_convert_type(
        ((xi + 0x8000) & jnp.uint32(0xFFFF0000)) | ((yi + 0x8000) >> 16),
        jnp.int32).reshape(B * N)
    zflat = xyz[:, :, 2].reshape(B * N)

    xyz3 = jnp.transpose(xyz, (0, 2, 1))
    nbr_t = jnp.transpose(neighbors.astype(jnp.int32), (0, 2, 1))
    nbr_tail = jnp.pad(nbr_t[:, :, T0:], ((0, 0), (0, 0), (0, NPAD - N)))
    dist_t = jnp.transpose(distances, (0, 2, 1))

    gxy, gz = sc_gather(xyp, zflat, nbr_t, nbr_tail)
    out_t = _make_tc_assemble(B, N, K, NPAD)(xyz3, gxy, gz, dist_t)
    return jnp.transpose(out_t, (0, 1, 3, 2))
